# Initial kernel scaffold; baseline (speedup 1.0000x reference)
import jax
import jax.numpy as jnp
from jax.experimental import pallas as pl


def kernel(x, edge_index, W1, b1, W2, b2, W3, b3):
    raise NotImplementedError("write your pallas kernel here")


# speedup vs baseline: 5.6118x; 5.6118x over previous
import functools

import jax
import jax.numpy as jnp
from jax import lax
from jax.experimental import pallas as pl
from jax.experimental.pallas import tpu as pltpu
from jax.experimental.pallas import tpu_sc as plsc

N = 10000
E = 320000
D = 128
H = 128

NC = 2
NS = 16
NW = NC * NS

NPAD = 10240
RPT = NPAD // NS
EW = E // NW
EC = 80
NCH = EW // EC

RB = NPAD // NW
CB = 160

@functools.partial(
    pl.kernel,
    out_type=[
        jax.ShapeDtypeStruct((NC, NPAD, D), jnp.float32),
        jax.ShapeDtypeStruct((NC * NPAD,), jnp.float32),
    ],
    mesh=plsc.VectorSubcoreMesh(
        core_axis_name="c", subcore_axis_name="s", num_cores=NC,
        num_subcores=NS),
    scratch_types=[
        pltpu.VMEM((EC,), jnp.int32),
        pltpu.VMEM((EC,), jnp.int32),
        pltpu.VMEM((EC, D), jnp.float32),
        pltpu.VMEM((EC,), jnp.float32),
        pltpu.VMEM((RPT,), jnp.float32),
        pltpu.VMEM_SHARED((NPAD, D), jnp.float32),
        pltpu.VMEM_SHARED((NPAD,), jnp.float32),
        pltpu.SemaphoreType.DMA,
    ],
)
def _sc_accumulate(x_hbm, src_hbm, dst_hbm, psum_hbm, pdeg_hbm,
                   sidx, didx, rows, ones, dzero, acc, dacc, sem):
  cid = lax.axis_index("c")
  sid = lax.axis_index("s")
  wid = cid * NS + sid

  def _zrow(r, _):
    for c in range(D // 16):
      rows[r, pl.ds(c * 16, 16)] = jnp.zeros((16,), jnp.float32)
    return _
  lax.fori_loop(0, EC, _zrow, None)
  for k in range(RPT // EC):
    pltpu.sync_copy(rows, acc.at[pl.ds(sid * RPT + k * EC, EC)])

  def _zdeg(i, _):
    dzero[pl.ds(i * 16, 16)] = jnp.zeros((16,), jnp.float32)
    return _
  lax.fori_loop(0, RPT // 16, _zdeg, None)
  pltpu.sync_copy(dzero, dacc.at[pl.ds(sid * RPT, RPT)])

  for i in range(EC // 16):
    ones[pl.ds(i * 16, 16)] = jnp.ones((16,), jnp.float32)

  plsc.subcore_barrier()

  base = wid * EW

  def _chunk(i, _):
    off = base + i * EC
    pltpu.sync_copy(src_hbm.at[pl.ds(off, EC)], sidx)
    pltpu.sync_copy(dst_hbm.at[pl.ds(off, EC)], didx)
    pltpu.async_copy(x_hbm.at[sidx], rows, sem).wait()
    pltpu.sync_copy(rows, acc.at[didx], add=True)
    pltpu.sync_copy(ones, dacc.at[didx], add=True)
    return _
  lax.fori_loop(0, NCH, _chunk, None)

  plsc.subcore_barrier()

  sl = pl.ds(sid * RPT, RPT)
  pltpu.sync_copy(acc.at[sl], psum_hbm.at[cid, sl])
  pltpu.sync_copy(dacc.at[sl], pdeg_hbm.at[pl.ds(cid * NPAD + sid * RPT, RPT)])


@functools.partial(
    pl.kernel,
    out_type=jax.ShapeDtypeStruct((NPAD, D), jnp.float32),
    mesh=plsc.VectorSubcoreMesh(
        core_axis_name="c", subcore_axis_name="s", num_cores=NC,
        num_subcores=NS),
    scratch_types=[
        pltpu.VMEM((CB, D), jnp.float32),
        pltpu.VMEM((CB, D), jnp.float32),
        pltpu.VMEM((CB, D), jnp.float32),
        pltpu.VMEM((CB, D), jnp.float32),
        pltpu.VMEM((RB,), jnp.float32),
        pltpu.VMEM((RB,), jnp.float32),
    ],
)
def _sc_combine(x_hbm, psum_hbm, pdeg_hbm, agg_hbm,
                xb, p0b, p1b, ob, d0b, d1b):
  cid = lax.axis_index("c")
  sid = lax.axis_index("s")
  wid = cid * NS + sid
  base = wid * RB

  pltpu.sync_copy(pdeg_hbm.at[pl.ds(base, RB)], d0b)
  pltpu.sync_copy(pdeg_hbm.at[pl.ds(NPAD + base, RB)], d1b)

  for ch in range(RB // CB):
    rb = base + ch * CB
    pltpu.sync_copy(x_hbm.at[pl.ds(rb, CB)], xb)
    pltpu.sync_copy(psum_hbm.at[0, pl.ds(rb, CB)], p0b)
    pltpu.sync_copy(psum_hbm.at[1, pl.ds(rb, CB)], p1b)

    def _rowgrp(g, _):
      d = d0b[pl.ds(ch * CB + g * 16, 16)] + d1b[pl.ds(ch * CB + g * 16, 16)]
      has = d > 0.0
      sn = jnp.where(has, 0.5 / jnp.maximum(d, 1.0), 0.0)
      sx = jnp.where(has, 0.5, 1.0)
      for j in range(16):
        snj = sn[j]
        sxj = sx[j]
        for c in range(D // 16):
          sl = pl.ds(c * 16, 16)
          r = g * 16 + j
          ob[r, sl] = xb[r, sl] * sxj + (p0b[r, sl] + p1b[r, sl]) * snj
      return _
    lax.fori_loop(0, CB // 16, _rowgrp, None)

    pltpu.sync_copy(ob, agg_hbm.at[pl.ds(rb, CB)])


BN = 2000


def _mlp_body(a_ref, w1_ref, b1_ref, w2_ref, b2_ref, w3_ref, b3_ref, o_ref):
  dn = (((1,), (1,)), ((), ()))
  a = a_ref[...]
  h = lax.dot_general(a, w1_ref[...], dn, preferred_element_type=jnp.float32)
  h = jnp.maximum(h + b1_ref[...], 0.0)
  h = lax.dot_general(h, w2_ref[...], dn, preferred_element_type=jnp.float32)
  h = jnp.maximum(h + b2_ref[...], 0.0)
  o_ref[...] = jnp.sum(h * w3_ref[...], axis=1, keepdims=True) + b3_ref[...]


def _tc_mlp(agg, W1, b1, W2, b2, W3, b3):
  return pl.pallas_call(
      _mlp_body,
      grid=(N // BN,),
      in_specs=[
          pl.BlockSpec((BN, D), lambda g: (g, 0)),
          pl.BlockSpec((H, D), lambda g: (0, 0)),
          pl.BlockSpec((1, H), lambda g: (0, 0)),
          pl.BlockSpec((H, H), lambda g: (0, 0)),
          pl.BlockSpec((1, H), lambda g: (0, 0)),
          pl.BlockSpec((1, H), lambda g: (0, 0)),
          pl.BlockSpec((1, 1), lambda g: (0, 0)),
      ],
      out_specs=pl.BlockSpec((BN, 1), lambda g: (g, 0)),
      out_shape=jax.ShapeDtypeStruct((N, 1), jnp.float32),
  )(agg, W1, b1.reshape(1, H), W2, b2.reshape(1, H), W3, b3.reshape(1, 1))


def kernel(x, edge_index, W1, b1, W2, b2, W3, b3):
  src = edge_index[0]
  dst = edge_index[1]
  x_pad = jnp.pad(x, ((0, NPAD - N), (0, 0)))
  psum, pdeg = _sc_accumulate(x_pad, src, dst)
  agg = _sc_combine(x_pad, psum, pdeg)
  return _tc_mlp(agg[:N], W1, b1, W2, b2, W3, b3)


# speedup vs baseline: 9.1528x; 1.6310x over previous
import functools

import jax
import jax.numpy as jnp
from jax import lax
from jax.experimental import pallas as pl
from jax.experimental.pallas import tpu as pltpu
from jax.experimental.pallas import tpu_sc as plsc

N = 10000
E = 320000
D = 128
H = 128

NC = 2
NS = 16
NW = NC * NS

NPAD = 10240
RPT = NPAD // NS
EW = E // NW
EC = 80
NCH = EW // EC

RB = NPAD // NW
CB = 160

@functools.partial(
    pl.kernel,
    out_type=[
        jax.ShapeDtypeStruct((NC, NPAD, D), jnp.float32),
        jax.ShapeDtypeStruct((NC * NPAD,), jnp.float32),
    ],
    mesh=plsc.VectorSubcoreMesh(
        core_axis_name="c", subcore_axis_name="s", num_cores=NC,
        num_subcores=NS),
    scratch_types=[
        pltpu.VMEM((EW,), jnp.int32),
        pltpu.VMEM((NCH, EC), jnp.int32),
        pltpu.VMEM((EC, D), jnp.float32),
        pltpu.VMEM((EC, D), jnp.float32),
        pltpu.VMEM((EC,), jnp.float32),
        pltpu.VMEM((RPT,), jnp.float32),
        pltpu.VMEM_SHARED((NPAD, D), jnp.float32),
        pltpu.VMEM_SHARED((NPAD,), jnp.float32),
        pltpu.SemaphoreType.DMA,
        pltpu.SemaphoreType.DMA,
        pltpu.SemaphoreType.DMA,
        pltpu.SemaphoreType.DMA,
        pltpu.SemaphoreType.DMA,
    ],
)
def _sc_accumulate(x_hbm, src_hbm, dst_hbm, psum_hbm, pdeg_hbm,
                   sbuf, dbuf, rows0, rows1, ones, dzero, acc, dacc,
                   gsem0, gsem1, ssem0, ssem1, dsem):
  cid = lax.axis_index("c")
  sid = lax.axis_index("s")
  wid = cid * NS + sid

  pltpu.sync_copy(src_hbm.at[pl.ds(wid * EW, EW)], sbuf)
  pltpu.sync_copy(dst_hbm.at[wid], dbuf)

  def _zrow(r, _):
    for c in range(D // 16):
      rows0[r, pl.ds(c * 16, 16)] = jnp.zeros((16,), jnp.float32)
    return _
  lax.fori_loop(0, EC, _zrow, None)
  for k in range(RPT // EC):
    pltpu.sync_copy(rows0, acc.at[pl.ds(sid * RPT + k * EC, EC)])

  def _zdeg(i, _):
    dzero[pl.ds(i * 16, 16)] = jnp.zeros((16,), jnp.float32)
    return _
  lax.fori_loop(0, RPT // 16, _zdeg, None)
  pltpu.sync_copy(dzero, dacc.at[pl.ds(sid * RPT, RPT)])

  for i in range(EC // 16):
    ones[pl.ds(i * 16, 16)] = jnp.ones((16,), jnp.float32)

  plsc.subcore_barrier()

  bufs = ((rows0, gsem0, ssem0), (rows1, gsem1, ssem1))

  pltpu.async_copy(x_hbm.at[sbuf.at[pl.ds(0, EC)]], rows0, gsem0)

  def _step(i, b):
    rows, gsem, ssem = bufs[b]
    pltpu.make_async_copy(x_hbm.at[sbuf.at[pl.ds(i * EC, EC)]], rows,
                          gsem).wait()
    pltpu.async_copy(rows, acc.at[dbuf.at[i]], ssem, add=True)
    pltpu.async_copy(ones, dacc.at[dbuf.at[i]], dsem, add=True)

    other_rows, other_gsem, other_ssem = bufs[1 - b]

    @pl.when(i > 0)
    def _():
      pltpu.make_async_copy(other_rows, acc.at[dbuf.at[i]], other_ssem).wait()
      pltpu.make_async_copy(ones, dacc.at[dbuf.at[i]], dsem).wait()

    @pl.when(i + 1 < NCH)
    def _():
      pltpu.async_copy(x_hbm.at[sbuf.at[pl.ds((i + 1) * EC, EC)]],
                       other_rows, other_gsem)

  def _pair(g, _):
    _step(2 * g, 0)
    _step(2 * g + 1, 1)
    return _
  lax.fori_loop(0, NCH // 2, _pair, None)
  _step(NCH - 1, 0)

  pltpu.make_async_copy(rows0, acc.at[dbuf.at[0]], ssem0).wait()
  pltpu.make_async_copy(ones, dacc.at[dbuf.at[0]], dsem).wait()

  plsc.subcore_barrier()

  sl = pl.ds(sid * RPT, RPT)
  pltpu.sync_copy(acc.at[sl], psum_hbm.at[cid, sl])
  pltpu.sync_copy(dacc.at[sl], pdeg_hbm.at[pl.ds(cid * NPAD + sid * RPT, RPT)])


@functools.partial(
    pl.kernel,
    out_type=jax.ShapeDtypeStruct((NPAD, D), jnp.float32),
    mesh=plsc.VectorSubcoreMesh(
        core_axis_name="c", subcore_axis_name="s", num_cores=NC,
        num_subcores=NS),
    scratch_types=[
        pltpu.VMEM((CB, D), jnp.float32),
        pltpu.VMEM((CB, D), jnp.float32),
        pltpu.VMEM((CB, D), jnp.float32),
        pltpu.VMEM((CB, D), jnp.float32),
        pltpu.VMEM((RB,), jnp.float32),
        pltpu.VMEM((RB,), jnp.float32),
    ],
)
def _sc_combine(x_hbm, psum_hbm, pdeg_hbm, agg_hbm,
                xb, p0b, p1b, ob, d0b, d1b):
  cid = lax.axis_index("c")
  sid = lax.axis_index("s")
  wid = cid * NS + sid
  base = wid * RB

  pltpu.sync_copy(pdeg_hbm.at[pl.ds(base, RB)], d0b)
  pltpu.sync_copy(pdeg_hbm.at[pl.ds(NPAD + base, RB)], d1b)

  for ch in range(RB // CB):
    rb = base + ch * CB
    pltpu.sync_copy(x_hbm.at[pl.ds(rb, CB)], xb)
    pltpu.sync_copy(psum_hbm.at[0, pl.ds(rb, CB)], p0b)
    pltpu.sync_copy(psum_hbm.at[1, pl.ds(rb, CB)], p1b)

    def _rowgrp(g, _):
      d = d0b[pl.ds(ch * CB + g * 16, 16)] + d1b[pl.ds(ch * CB + g * 16, 16)]
      has = d > 0.0
      sn = jnp.where(has, 0.5 / jnp.maximum(d, 1.0), 0.0)
      sx = jnp.where(has, 0.5, 1.0)
      for j in range(16):
        snj = sn[j]
        sxj = sx[j]
        for c in range(D // 16):
          sl = pl.ds(c * 16, 16)
          r = g * 16 + j
          ob[r, sl] = xb[r, sl] * sxj + (p0b[r, sl] + p1b[r, sl]) * snj
      return _
    lax.fori_loop(0, CB // 16, _rowgrp, None)

    pltpu.sync_copy(ob, agg_hbm.at[pl.ds(rb, CB)])


BN = 2000


def _mlp_body(a_ref, w1_ref, b1_ref, w2_ref, b2_ref, w3_ref, b3_ref, o_ref):
  dn = (((1,), (1,)), ((), ()))
  a = a_ref[...]
  h = lax.dot_general(a, w1_ref[...], dn, preferred_element_type=jnp.float32)
  h = jnp.maximum(h + b1_ref[...], 0.0)
  h = lax.dot_general(h, w2_ref[...], dn, preferred_element_type=jnp.float32)
  h = jnp.maximum(h + b2_ref[...], 0.0)
  o_ref[...] = jnp.sum(h * w3_ref[...], axis=1, keepdims=True) + b3_ref[...]


def _tc_mlp(agg, W1, b1, W2, b2, W3, b3):
  return pl.pallas_call(
      _mlp_body,
      grid=(N // BN,),
      in_specs=[
          pl.BlockSpec((BN, D), lambda g: (g, 0)),
          pl.BlockSpec((H, D), lambda g: (0, 0)),
          pl.BlockSpec((1, H), lambda g: (0, 0)),
          pl.BlockSpec((H, H), lambda g: (0, 0)),
          pl.BlockSpec((1, H), lambda g: (0, 0)),
          pl.BlockSpec((1, H), lambda g: (0, 0)),
          pl.BlockSpec((1, 1), lambda g: (0, 0)),
      ],
      out_specs=pl.BlockSpec((BN, 1), lambda g: (g, 0)),
      out_shape=jax.ShapeDtypeStruct((N, 1), jnp.float32),
  )(agg, W1, b1.reshape(1, H), W2, b2.reshape(1, H), W3, b3.reshape(1, 1))


def kernel(x, edge_index, W1, b1, W2, b2, W3, b3):
  src = edge_index[0]
  dst = edge_index[1].reshape(NW, NCH, EC)
  x_pad = jnp.pad(x, ((0, NPAD - N), (0, 0)))
  psum, pdeg = _sc_accumulate(x_pad, src, dst)
  agg = _sc_combine(x_pad, psum, pdeg)
  return _tc_mlp(agg[:N], W1, b1, W2, b2, W3, b3)
